# baseline (speedup 1.0000x reference)
import jax
import jax.numpy as jnp
from jax.experimental import pallas as pl
from jax.experimental.pallas import tpu as pltpu

_N = 10000
_E = 2000
_D = 128
_NB = 1000
_GRID = _N // _NB


def _intra1_kernel(x_ref, inc_ref, out_ref, dege_ref, acc_ref, dacc_ref):
    i = pl.program_id(0)
    inc = inc_ref[:]
    v = x_ref[:]
    yT = jnp.transpose(v * v).astype(jnp.bfloat16)
    part = jax.lax.dot_general(
        yT, inc, (((1,), (0,)), ((), ())),
        preferred_element_type=jnp.float32)
    dpart = jnp.sum(inc, axis=0, keepdims=True, dtype=jnp.float32)

    @pl.when(i == 0)
    def _init():
        acc_ref[:] = part
        dacc_ref[:] = dpart

    @pl.when(i > 0)
    def _accum():
        acc_ref[:] += part
        dacc_ref[:] += dpart

    @pl.when(i == _GRID - 1)
    def _finish():
        deg = jnp.maximum(dacc_ref[:], 1.0)
        dege_ref[:] = deg
        out_ref[:] = jnp.transpose(acc_ref[:] / deg).astype(jnp.bfloat16)


def _fused_kernel(inc_ref, intra_ref, w_ref, dege_ref,
                  out_ref, degv_ref, acc_ref):
    i = pl.program_id(0)
    inc = inc_ref[:]
    s2 = jax.lax.dot_general(
        inc, intra_ref[:], (((1,), (0,)), ((), ())),
        preferred_element_type=jnp.float32)
    dv = jnp.sum(inc, axis=1, keepdims=True, dtype=jnp.float32)
    dvf = jnp.maximum(dv, 1.0)
    degv_ref[:] = dvf
    inter = jnp.sqrt(s2 / dvf)
    msg = jnp.dot(inter, w_ref[:], preferred_element_type=jnp.float32)
    act = jnp.maximum(msg, 0.0)
    asqT = jnp.transpose(act * act).astype(jnp.bfloat16)
    part = jax.lax.dot_general(
        asqT, inc, (((1,), (0,)), ((), ())),
        preferred_element_type=jnp.float32)

    @pl.when(i == 0)
    def _init():
        acc_ref[:] = part

    @pl.when(i > 0)
    def _accum():
        acc_ref[:] += part

    @pl.when(i == _GRID - 1)
    def _finish():
        out_ref[:] = jnp.transpose(
            acc_ref[:] / dege_ref[:]).astype(jnp.bfloat16)


def _inter2_kernel(inc_ref, intra_ref, w_ref, degv_ref, out_ref):
    s2 = jax.lax.dot_general(
        inc_ref[:], intra_ref[:], (((1,), (0,)), ((), ())),
        preferred_element_type=jnp.float32)
    inter = jnp.sqrt(s2 / degv_ref[:])
    msg = jnp.dot(inter, w_ref[:], preferred_element_type=jnp.float32)
    out_ref[:] = jnp.maximum(msg, 0.0)


def kernel(x_0, incidence_1, W1, W2):
    inc_bf = incidence_1.astype(jnp.bfloat16)

    intra1, deg_e = pl.pallas_call(
        _intra1_kernel,
        grid=(_GRID,),
        in_specs=[
            pl.BlockSpec((_NB, _D), lambda i: (i, 0)),
            pl.BlockSpec((_NB, _E), lambda i: (i, 0)),
        ],
        out_specs=[
            pl.BlockSpec((_E, _D), lambda i: (0, 0)),
            pl.BlockSpec((1, _E), lambda i: (0, 0)),
        ],
        out_shape=[
            jax.ShapeDtypeStruct((_E, _D), jnp.bfloat16),
            jax.ShapeDtypeStruct((1, _E), jnp.float32),
        ],
        scratch_shapes=[
            pltpu.VMEM((_D, _E), jnp.float32),
            pltpu.VMEM((1, _E), jnp.float32),
        ],
    )(x_0, inc_bf)

    intra2, deg_v = pl.pallas_call(
        _fused_kernel,
        grid=(_GRID,),
        in_specs=[
            pl.BlockSpec((_NB, _E), lambda i: (i, 0)),
            pl.BlockSpec((_E, _D), lambda i: (0, 0)),
            pl.BlockSpec((_D, _D), lambda i: (0, 0)),
            pl.BlockSpec((1, _E), lambda i: (0, 0)),
        ],
        out_specs=[
            pl.BlockSpec((_E, _D), lambda i: (0, 0)),
            pl.BlockSpec((_NB, 1), lambda i: (i, 0)),
        ],
        out_shape=[
            jax.ShapeDtypeStruct((_E, _D), jnp.bfloat16),
            jax.ShapeDtypeStruct((_N, 1), jnp.float32),
        ],
        scratch_shapes=[pltpu.VMEM((_D, _E), jnp.float32)],
    )(inc_bf, intra1, W1, deg_e)

    out = pl.pallas_call(
        _inter2_kernel,
        grid=(_GRID,),
        in_specs=[
            pl.BlockSpec((_NB, _E), lambda i: (i, 0)),
            pl.BlockSpec((_E, _D), lambda i: (0, 0)),
            pl.BlockSpec((_D, _D), lambda i: (0, 0)),
            pl.BlockSpec((_NB, 1), lambda i: (i, 0)),
        ],
        out_specs=pl.BlockSpec((_NB, _D), lambda i: (i, 0)),
        out_shape=jax.ShapeDtypeStruct((_N, _D), jnp.float32),
    )(inc_bf, intra2, W2, deg_v)

    return out


# speedup vs baseline: 1.0242x; 1.0242x over previous
import jax
import jax.numpy as jnp
from jax.experimental import pallas as pl
from jax.experimental.pallas import tpu as pltpu

_N = 10000
_E = 2000
_D = 128
_NB = 2000
_GRID = _N // _NB


def _intra1_kernel(x_ref, inc_ref, out_ref, dege_ref, acc_ref, dacc_ref):
    i = pl.program_id(0)
    inc = inc_ref[:]
    v = x_ref[:]
    yT = jnp.transpose(v * v).astype(jnp.bfloat16)
    part = jax.lax.dot_general(
        yT, inc, (((1,), (0,)), ((), ())),
        preferred_element_type=jnp.float32)
    dpart = jnp.sum(inc, axis=0, keepdims=True, dtype=jnp.float32)

    @pl.when(i == 0)
    def _init():
        acc_ref[:] = part
        dacc_ref[:] = dpart

    @pl.when(i > 0)
    def _accum():
        acc_ref[:] += part
        dacc_ref[:] += dpart

    @pl.when(i == _GRID - 1)
    def _finish():
        deg = jnp.maximum(dacc_ref[:], 1.0)
        dege_ref[:] = deg
        out_ref[:] = jnp.transpose(acc_ref[:] / deg).astype(jnp.bfloat16)


def _fused_kernel(inc_ref, intra_ref, w_ref, dege_ref,
                  out_ref, degv_ref, acc_ref):
    i = pl.program_id(0)
    inc = inc_ref[:]
    s2 = jax.lax.dot_general(
        inc, intra_ref[:], (((1,), (0,)), ((), ())),
        preferred_element_type=jnp.float32)
    dv = jnp.sum(inc, axis=1, keepdims=True, dtype=jnp.float32)
    dvf = jnp.maximum(dv, 1.0)
    degv_ref[:] = dvf
    inter = jnp.sqrt(s2 / dvf)
    msg = jnp.dot(inter, w_ref[:], preferred_element_type=jnp.float32)
    act = jnp.maximum(msg, 0.0)
    asqT = jnp.transpose(act * act).astype(jnp.bfloat16)
    part = jax.lax.dot_general(
        asqT, inc, (((1,), (0,)), ((), ())),
        preferred_element_type=jnp.float32)

    @pl.when(i == 0)
    def _init():
        acc_ref[:] = part

    @pl.when(i > 0)
    def _accum():
        acc_ref[:] += part

    @pl.when(i == _GRID - 1)
    def _finish():
        out_ref[:] = jnp.transpose(
            acc_ref[:] / dege_ref[:]).astype(jnp.bfloat16)


def _inter2_kernel(inc_ref, intra_ref, w_ref, degv_ref, out_ref):
    s2 = jax.lax.dot_general(
        inc_ref[:], intra_ref[:], (((1,), (0,)), ((), ())),
        preferred_element_type=jnp.float32)
    inter = jnp.sqrt(s2 / degv_ref[:])
    msg = jnp.dot(inter, w_ref[:], preferred_element_type=jnp.float32)
    out_ref[:] = jnp.maximum(msg, 0.0)


def kernel(x_0, incidence_1, W1, W2):
    inc_bf = incidence_1.astype(jnp.bfloat16)

    intra1, deg_e = pl.pallas_call(
        _intra1_kernel,
        grid=(_GRID,),
        in_specs=[
            pl.BlockSpec((_NB, _D), lambda i: (i, 0)),
            pl.BlockSpec((_NB, _E), lambda i: (i, 0)),
        ],
        out_specs=[
            pl.BlockSpec((_E, _D), lambda i: (0, 0)),
            pl.BlockSpec((1, _E), lambda i: (0, 0)),
        ],
        out_shape=[
            jax.ShapeDtypeStruct((_E, _D), jnp.bfloat16),
            jax.ShapeDtypeStruct((1, _E), jnp.float32),
        ],
        scratch_shapes=[
            pltpu.VMEM((_D, _E), jnp.float32),
            pltpu.VMEM((1, _E), jnp.float32),
        ],
    )(x_0, inc_bf)

    intra2, deg_v = pl.pallas_call(
        _fused_kernel,
        grid=(_GRID,),
        in_specs=[
            pl.BlockSpec((_NB, _E), lambda i: (i, 0)),
            pl.BlockSpec((_E, _D), lambda i: (0, 0)),
            pl.BlockSpec((_D, _D), lambda i: (0, 0)),
            pl.BlockSpec((1, _E), lambda i: (0, 0)),
        ],
        out_specs=[
            pl.BlockSpec((_E, _D), lambda i: (0, 0)),
            pl.BlockSpec((_NB, 1), lambda i: (i, 0)),
        ],
        out_shape=[
            jax.ShapeDtypeStruct((_E, _D), jnp.bfloat16),
            jax.ShapeDtypeStruct((_N, 1), jnp.float32),
        ],
        scratch_shapes=[pltpu.VMEM((_D, _E), jnp.float32)],
    )(inc_bf, intra1, W1, deg_e)

    out = pl.pallas_call(
        _inter2_kernel,
        grid=(_GRID,),
        in_specs=[
            pl.BlockSpec((_NB, _E), lambda i: (i, 0)),
            pl.BlockSpec((_E, _D), lambda i: (0, 0)),
            pl.BlockSpec((_D, _D), lambda i: (0, 0)),
            pl.BlockSpec((_NB, 1), lambda i: (i, 0)),
        ],
        out_specs=pl.BlockSpec((_NB, _D), lambda i: (i, 0)),
        out_shape=jax.ShapeDtypeStruct((_N, _D), jnp.float32),
    )(inc_bf, intra2, W2, deg_v)

    return out
